# speedup vs baseline: 2.6288x; 2.6288x over previous
import functools

import jax
import jax.numpy as jnp
from jax.experimental import pallas as pl
from jax.experimental.pallas import tpu as pltpu

_B = 2
_S = 2048
_H = 2048
_E = 8
_A = 128
_T = _B * _S
_TILE = 512


def _fused_body(xi_ref, xo_ref, xr_ref, wr_ref, wd_ref, wu_ref, out_ref, lg_ref):
    logits = jax.lax.dot_general(
        xr_ref[...], wr_ref[...], (((1,), (0,)), ((), ())),
        precision=jax.lax.Precision.HIGHEST,
        preferred_element_type=jnp.float32)
    lg_ref[...] = logits

    p = jax.nn.softmax(logits, axis=-1)

    cols = jax.lax.broadcasted_iota(jnp.int32, p.shape, 1)
    p1 = jnp.max(p, axis=-1, keepdims=True)
    i1 = jnp.argmax(p, axis=-1, keepdims=True)
    m1 = cols == i1
    p_rest = jnp.where(m1, -jnp.inf, p)
    p2 = jnp.max(p_rest, axis=-1, keepdims=True)
    i2 = jnp.argmax(p_rest, axis=-1, keepdims=True)
    m2 = cols == i2
    w = jnp.where(m1 | m2, p, 0.0) / (p1 + p2)

    xi = xi_ref[...].astype(jnp.bfloat16)
    h = jax.lax.dot_general(
        xi, wd_ref[...], (((1,), (0,)), ((), ())),
        preferred_element_type=jnp.float32)
    h = jax.nn.gelu(h)
    hw = h.reshape(_TILE, _E, _A) * w[:, :, None]
    hw = hw.reshape(_TILE, _E * _A).astype(jnp.bfloat16)
    res = jax.lax.dot_general(
        hw, wu_ref[...], (((1,), (0,)), ((), ())),
        preferred_element_type=jnp.float32)
    out_ref[...] = xo_ref[...] + res


@jax.jit
def kernel(input_hidden_states, output_hidden_states, router_hidden_states,
           W_router, W_down, W_up):
    orig_shape = output_hidden_states.shape
    xi = input_hidden_states.reshape(_T, _H)
    xo = output_hidden_states.reshape(_T, _H)
    xr = router_hidden_states.reshape(_T, _H)
    wr = W_router.T
    wd = W_down.transpose(2, 0, 1).reshape(_H, _E * _A).astype(jnp.bfloat16)
    wu = W_up.transpose(0, 2, 1).reshape(_E * _A, _H).astype(jnp.bfloat16)

    grid = (_T // _TILE,)
    out, logits = pl.pallas_call(
        _fused_body,
        grid=grid,
        in_specs=[
            pl.BlockSpec((_TILE, _H), lambda i: (i, 0)),
            pl.BlockSpec((_TILE, _H), lambda i: (i, 0)),
            pl.BlockSpec((_TILE, _H), lambda i: (i, 0)),
            pl.BlockSpec((_H, _E), lambda i: (0, 0)),
            pl.BlockSpec((_H, _E * _A), lambda i: (0, 0)),
            pl.BlockSpec((_E * _A, _H), lambda i: (0, 0)),
        ],
        out_specs=[
            pl.BlockSpec((_TILE, _H), lambda i: (i, 0)),
            pl.BlockSpec((_TILE, _E), lambda i: (i, 0)),
        ],
        out_shape=[
            jax.ShapeDtypeStruct((_T, _H), jnp.float32),
            jax.ShapeDtypeStruct((_T, _E), jnp.float32),
        ],
        compiler_params=pltpu.CompilerParams(
            dimension_semantics=("parallel",),
        ),
    )(xi, xo, xr, wr, wd, wu)

    return out.reshape(orig_shape), logits


# speedup vs baseline: 3.4109x; 1.2975x over previous
import functools

import jax
import jax.numpy as jnp
from jax.experimental import pallas as pl
from jax.experimental.pallas import tpu as pltpu

_B = 2
_S = 2048
_H = 2048
_E = 8
_A = 128
_T = _B * _S
_TILE = 512


def _dot_bf16(a, b):
    return jax.lax.dot_general(
        a, b, (((1,), (0,)), ((), ())), preferred_element_type=jnp.float32)


def _fused_body(xi_ref, xo_ref, xr_ref, wrh_ref, wrl_ref, wd_ref, wu_ref,
                out_ref, lg_ref):
    xr = xr_ref[...]
    xr_hi = xr.astype(jnp.bfloat16)
    xr_lo = (xr - xr_hi.astype(jnp.float32)).astype(jnp.bfloat16)
    logits = (_dot_bf16(xr_hi, wrh_ref[...])
              + (_dot_bf16(xr_lo, wrh_ref[...])
                 + _dot_bf16(xr_hi, wrl_ref[...])))
    lg_ref[...] = logits

    p = jax.nn.softmax(logits, axis=-1)

    cols = jax.lax.broadcasted_iota(jnp.int32, p.shape, 1)
    p1 = jnp.max(p, axis=-1, keepdims=True)
    i1 = jnp.argmax(p, axis=-1, keepdims=True)
    m1 = cols == i1
    p_rest = jnp.where(m1, -jnp.inf, p)
    p2 = jnp.max(p_rest, axis=-1, keepdims=True)
    i2 = jnp.argmax(p_rest, axis=-1, keepdims=True)
    m2 = cols == i2
    w = jnp.where(m1 | m2, p, 0.0) / (p1 + p2)

    xi = xi_ref[...].astype(jnp.bfloat16)
    h = jax.lax.dot_general(
        xi, wd_ref[...], (((1,), (0,)), ((), ())),
        preferred_element_type=jnp.float32)
    h = jax.nn.gelu(h.astype(jnp.bfloat16))
    wb = w.astype(jnp.bfloat16)
    hw = (h.reshape(_TILE, _E, _A) * wb[:, :, None]).reshape(_TILE, _E * _A)
    res = jax.lax.dot_general(
        hw, wu_ref[...], (((1,), (0,)), ((), ())),
        preferred_element_type=jnp.float32)
    out_ref[...] = xo_ref[...] + res


@jax.jit
def kernel(input_hidden_states, output_hidden_states, router_hidden_states,
           W_router, W_down, W_up):
    orig_shape = output_hidden_states.shape
    xi = input_hidden_states.reshape(_T, _H)
    xo = output_hidden_states.reshape(_T, _H)
    xr = router_hidden_states.reshape(_T, _H)
    wr = W_router.T
    wr_hi = wr.astype(jnp.bfloat16)
    wr_lo = (wr - wr_hi.astype(jnp.float32)).astype(jnp.bfloat16)
    wd = W_down.transpose(2, 0, 1).reshape(_H, _E * _A).astype(jnp.bfloat16)
    wu = W_up.transpose(0, 2, 1).reshape(_E * _A, _H).astype(jnp.bfloat16)

    grid = (_T // _TILE,)
    out, logits = pl.pallas_call(
        _fused_body,
        grid=grid,
        in_specs=[
            pl.BlockSpec((_TILE, _H), lambda i: (i, 0)),
            pl.BlockSpec((_TILE, _H), lambda i: (i, 0)),
            pl.BlockSpec((_TILE, _H), lambda i: (i, 0)),
            pl.BlockSpec((_H, _E), lambda i: (0, 0)),
            pl.BlockSpec((_H, _E), lambda i: (0, 0)),
            pl.BlockSpec((_H, _E * _A), lambda i: (0, 0)),
            pl.BlockSpec((_E * _A, _H), lambda i: (0, 0)),
        ],
        out_specs=[
            pl.BlockSpec((_TILE, _H), lambda i: (i, 0)),
            pl.BlockSpec((_TILE, _E), lambda i: (i, 0)),
        ],
        out_shape=[
            jax.ShapeDtypeStruct((_T, _H), jnp.float32),
            jax.ShapeDtypeStruct((_T, _E), jnp.float32),
        ],
        compiler_params=pltpu.CompilerParams(
            dimension_semantics=("parallel",),
        ),
    )(xi, xo, xr, wr_hi, wr_lo, wd, wu)

    return out.reshape(orig_shape), logits


# speedup vs baseline: 3.8121x; 1.1176x over previous
import functools

import jax
import jax.numpy as jnp
from jax.experimental import pallas as pl
from jax.experimental.pallas import tpu as pltpu

_B = 2
_S = 2048
_H = 2048
_E = 8
_A = 128
_T = _B * _S
_TILE = 512


def _dot_bf16(a, b):
    return jax.lax.dot_general(
        a, b, (((1,), (0,)), ((), ())), preferred_element_type=jnp.float32)


def _fused_body(xi_ref, xo_ref, xr_ref, wrh_ref, wrl_ref, wd_ref, wu_ref,
                exp_ref, out_ref, lg_ref):
    xr = xr_ref[...]
    xr_hi = xr.astype(jnp.bfloat16)
    xr_lo = (xr - xr_hi.astype(jnp.float32)).astype(jnp.bfloat16)
    logits = (_dot_bf16(xr_hi, wrh_ref[...])
              + (_dot_bf16(xr_lo, wrh_ref[...])
                 + _dot_bf16(xr_hi, wrl_ref[...])))
    lg_ref[...] = logits

    cols = jax.lax.broadcasted_iota(jnp.int32, logits.shape, 1)
    l1 = jnp.max(logits, axis=-1, keepdims=True)
    i1 = jnp.argmax(logits, axis=-1, keepdims=True)
    m1 = cols == i1
    l_rest = jnp.where(m1, -jnp.inf, logits)
    l2 = jnp.max(l_rest, axis=-1, keepdims=True)
    i2 = jnp.argmax(l_rest, axis=-1, keepdims=True)
    m2 = cols == i2
    e2 = jnp.exp(l2 - l1)
    w1 = 1.0 / (1.0 + e2)
    w = jnp.where(m1, w1, jnp.where(m2, e2 * w1, 0.0))

    xi = xi_ref[...].astype(jnp.bfloat16)
    h = jax.lax.dot_general(
        xi, wd_ref[...], (((1,), (0,)), ((), ())),
        preferred_element_type=jnp.float32)
    h = jax.nn.gelu(h.astype(jnp.bfloat16))
    w_exp = jax.lax.dot_general(
        w.astype(jnp.bfloat16), exp_ref[...], (((1,), (0,)), ((), ())),
        preferred_element_type=jnp.float32).astype(jnp.bfloat16)
    hw = h * w_exp
    res = jax.lax.dot_general(
        hw, wu_ref[...], (((1,), (0,)), ((), ())),
        preferred_element_type=jnp.float32)
    out_ref[...] = xo_ref[...] + res


@jax.jit
def kernel(input_hidden_states, output_hidden_states, router_hidden_states,
           W_router, W_down, W_up):
    orig_shape = output_hidden_states.shape
    xi = input_hidden_states.reshape(_T, _H)
    xo = output_hidden_states.reshape(_T, _H)
    xr = router_hidden_states.reshape(_T, _H)
    wr = W_router.T
    wr_hi = wr.astype(jnp.bfloat16)
    wr_lo = (wr - wr_hi.astype(jnp.float32)).astype(jnp.bfloat16)
    wd = W_down.transpose(2, 0, 1).reshape(_H, _E * _A).astype(jnp.bfloat16)
    wu = W_up.transpose(0, 2, 1).reshape(_E * _A, _H).astype(jnp.bfloat16)
    expand = jnp.repeat(jnp.eye(_E, dtype=jnp.bfloat16), _A, axis=1)

    grid = (_T // _TILE,)
    out, logits = pl.pallas_call(
        _fused_body,
        grid=grid,
        in_specs=[
            pl.BlockSpec((_TILE, _H), lambda i: (i, 0)),
            pl.BlockSpec((_TILE, _H), lambda i: (i, 0)),
            pl.BlockSpec((_TILE, _H), lambda i: (i, 0)),
            pl.BlockSpec((_H, _E), lambda i: (0, 0)),
            pl.BlockSpec((_H, _E), lambda i: (0, 0)),
            pl.BlockSpec((_H, _E * _A), lambda i: (0, 0)),
            pl.BlockSpec((_E * _A, _H), lambda i: (0, 0)),
            pl.BlockSpec((_E, _E * _A), lambda i: (0, 0)),
        ],
        out_specs=[
            pl.BlockSpec((_TILE, _H), lambda i: (i, 0)),
            pl.BlockSpec((_TILE, _E), lambda i: (i, 0)),
        ],
        out_shape=[
            jax.ShapeDtypeStruct((_T, _H), jnp.float32),
            jax.ShapeDtypeStruct((_T, _E), jnp.float32),
        ],
        compiler_params=pltpu.CompilerParams(
            dimension_semantics=("parallel",),
        ),
    )(xi, xo, xr, wr_hi, wr_lo, wd, wu, expand)

    return out.reshape(orig_shape), logits
